# baseline (speedup 1.0000x reference)
import functools

import jax
import jax.numpy as jnp
from jax import lax
from jax.experimental import pallas as pl
from jax.experimental.pallas import tpu as pltpu
from jax.experimental.pallas import tpu_sc as plsc

_B = 64
_D = 10000
_E = 320000
_G = 100
_GS = 100
_EPS = 1e-5

_NC = 2
_NS = 16
_NW = _NC * _NS
_CH = 128
_NCH0 = 72
_NCH1 = 88
_MAXCH = _NCH1
_EP = (_NS * (_NCH0 + _NCH1)) * _CH
_BAND = 640
_TAIL = _D - 15 * _BAND


def _band_copy(s, src, dst):
    @pl.when(s < _NS - 1)
    def _():
        pltpu.sync_copy(src.at[pl.ds(s * _BAND, _BAND)],
                        dst.at[pl.ds(s * _BAND, _BAND)])

    @pl.when(s == _NS - 1)
    def _():
        pltpu.sync_copy(src.at[pl.ds(15 * _BAND, _TAIL)],
                        dst.at[pl.ds(15 * _BAND, _TAIL)])


def _sc_linear_body(table, cols, rows, wvals, zeros, out, acc, eidx, ew, gbuf, gsem):
    c = lax.axis_index("c")
    s = lax.axis_index("s")
    wid = c * _NS + s
    _band_copy(s, zeros, acc)
    pltpu.sync_copy(cols.at[wid], eidx.at[0])
    pltpu.sync_copy(rows.at[wid], eidx.at[1])
    pltpu.sync_copy(wvals.at[wid], ew)
    plsc.subcore_barrier()

    def chunk(i, carry):
        pltpu.async_copy(table.at[eidx.at[0, i]], gbuf, gsem).wait()
        base_vec = jnp.full((16,), 0, jnp.int32) + i * _CH
        for j in range(_CH):
            wj = plsc.load_gather(ew, [base_vec + j])
            for k in range(_B // 16):
                v = gbuf[j, pl.ds(k * 16, 16)]
                gbuf[j, pl.ds(k * 16, 16)] = v * wj
        pltpu.sync_copy(gbuf, acc.at[eidx.at[1, i]], add=True)
        return carry

    ncheck = jnp.where(c == 0, _NCH0, _NCH1)
    lax.fori_loop(0, ncheck, chunk, 0)
    plsc.subcore_barrier()
    _band_copy(s, acc, out.at[c])


@functools.cache
def _get_sc_linear():
    return pl.kernel(
        _sc_linear_body,
        out_type=jax.ShapeDtypeStruct((_NC, _D, _B), jnp.float32),
        mesh=plsc.VectorSubcoreMesh(core_axis_name="c", subcore_axis_name="s",
                                    num_cores=_NC, num_subcores=_NS),
        compiler_params=pltpu.CompilerParams(needs_layout_passes=False,
                                             use_tc_tiling_on_sc=False),
        scratch_types=[
            pltpu.VMEM_SHARED((_D, _B), jnp.float32),
            pltpu.VMEM((2, _MAXCH, _CH), jnp.int32),
            pltpu.VMEM((_MAXCH * _CH,), jnp.float32),
            pltpu.VMEM((_CH, _B), jnp.float32),
            pltpu.SemaphoreType.DMA,
        ],
    )


_RG = 10


def _tc_norm_body(p_ref, b_ref, g_ref, be_ref, o_ref):
    acc = p_ref[0] + p_ref[1]
    acc = acc + b_ref[0][:, :, None]
    mu = jnp.mean(acc, axis=1, keepdims=True)
    xc = acc - mu
    var = jnp.mean(xc * xc, axis=1, keepdims=True)
    y = xc * lax.rsqrt(var + _EPS)
    y = y * g_ref[0][:, :, None] + be_ref[0][:, :, None]
    o_ref[...] = jnp.maximum(y, 0.0)


_tc_norm = pl.pallas_call(
    _tc_norm_body,
    grid=(_G // _RG,),
    in_specs=[
        pl.BlockSpec((2, _RG, _GS, _B), lambda i: (0, i, 0, 0)),
        pl.BlockSpec((1, _RG, _GS), lambda i: (i, 0, 0)),
        pl.BlockSpec((1, _RG, _GS), lambda i: (i, 0, 0)),
        pl.BlockSpec((1, _RG, _GS), lambda i: (i, 0, 0)),
    ],
    out_specs=pl.BlockSpec((_RG, _GS, _B), lambda i: (i, 0, 0)),
    out_shape=jax.ShapeDtypeStruct((_G, _GS, _B), jnp.float32),
)

_RROW = 1000


def _tc_final_body(p_ref, b_ref, x_ref, o_ref):
    o_ref[...] = p_ref[0] + p_ref[1] + b_ref[...] + x_ref[...]


_tc_final = pl.pallas_call(
    _tc_final_body,
    grid=(_D // _RROW,),
    in_specs=[
        pl.BlockSpec((2, _RROW, _B), lambda i: (0, i, 0)),
        pl.BlockSpec((_RROW, 1), lambda i: (i, 0)),
        pl.BlockSpec((_RROW, _B), lambda i: (i, 0)),
    ],
    out_specs=pl.BlockSpec((_RROW, _B), lambda i: (i, 0)),
    out_shape=jax.ShapeDtypeStruct((_D, _B), jnp.float32),
)


def _split_uneven(flat):
    n0 = _NS * _NCH0 * _CH
    c0 = flat[:n0].reshape(_NS, _NCH0 * _CH)
    c0 = jnp.pad(c0, ((0, 0), (0, (_MAXCH - _NCH0) * _CH)))
    c1 = flat[n0:].reshape(_NS, _NCH1 * _CH)
    return jnp.concatenate([c0, c1], axis=0)


def _prep_edges(ei, w):
    pad = _EP - _E
    r = _split_uneven(jnp.pad(ei[0], (0, pad))).reshape(_NW, _MAXCH, _CH)
    c = _split_uneven(jnp.pad(ei[1], (0, pad))).reshape(_NW, _MAXCH, _CH)
    wp = _split_uneven(jnp.pad(w, (0, pad)))
    return r, c, wp


def kernel(x, batched_edge_indices1, batched_edge_indices2, batched_edge_indices3,
           w1, b1, gamma1, beta1, w2, b2, gamma2, beta2, w3, b3):
    xT = x.T
    zeros = jnp.zeros((_D, _B), jnp.float32)
    r1, c1, wp1 = _prep_edges(batched_edge_indices1, w1)
    r2, c2, wp2 = _prep_edges(batched_edge_indices2, w2)
    r3, c3, wp3 = _prep_edges(batched_edge_indices3, w3)

    _sc_linear = _get_sc_linear()
    _shp = (_G // _RG, _RG, _GS)
    p1 = _sc_linear(xT, c1, r1, wp1, zeros)
    h1 = _tc_norm(p1.reshape(_NC, _G, _GS, _B), b1.reshape(_shp),
                  gamma1.reshape(_shp), beta1.reshape(_shp)).reshape(_D, _B)
    p2 = _sc_linear(h1, c2, r2, wp2, zeros)
    h2 = _tc_norm(p2.reshape(_NC, _G, _GS, _B), b2.reshape(_shp),
                  gamma2.reshape(_shp), beta2.reshape(_shp)).reshape(_D, _B)
    p3 = _sc_linear(h2, c3, r3, wp3, zeros)
    outT = _tc_final(p3, b3.reshape(_D, 1), xT)
    return outT.T


# speedup vs baseline: 1.0981x; 1.0981x over previous
import functools

import jax
import jax.numpy as jnp
from jax import lax
from jax.experimental import pallas as pl
from jax.experimental.pallas import tpu as pltpu
from jax.experimental.pallas import tpu_sc as plsc

_B = 64
_D = 10000
_E = 320000
_G = 100
_GS = 100
_EPS = 1e-5

_NC = 2
_NS = 16
_NW = _NC * _NS
_CH = 128
_NCH0 = 88
_NCH1 = 72
_MAXCH = _NCH0
_EP = (_NS * (_NCH0 + _NCH1)) * _CH
_BAND = 640
_TAIL = _D - 15 * _BAND


def _band_copy(s, src, dst):
    @pl.when(s < _NS - 1)
    def _():
        pltpu.sync_copy(src.at[pl.ds(s * _BAND, _BAND)],
                        dst.at[pl.ds(s * _BAND, _BAND)])

    @pl.when(s == _NS - 1)
    def _():
        pltpu.sync_copy(src.at[pl.ds(15 * _BAND, _TAIL)],
                        dst.at[pl.ds(15 * _BAND, _TAIL)])


def _sc_linear_body(table, cols, rows, wvals, zeros, out, acc, eidx, ew, gbuf, gsem):
    c = lax.axis_index("c")
    s = lax.axis_index("s")
    wid = c * _NS + s
    _band_copy(s, zeros, acc)
    pltpu.sync_copy(cols.at[wid], eidx.at[0])
    pltpu.sync_copy(rows.at[wid], eidx.at[1])
    pltpu.sync_copy(wvals.at[wid], ew)
    plsc.subcore_barrier()

    def chunk(i, carry):
        pltpu.async_copy(table.at[eidx.at[0, i]], gbuf, gsem).wait()
        base_vec = jnp.full((16,), 0, jnp.int32) + i * _CH
        for j in range(_CH):
            wj = plsc.load_gather(ew, [base_vec + j])
            for k in range(_B // 16):
                v = gbuf[j, pl.ds(k * 16, 16)]
                gbuf[j, pl.ds(k * 16, 16)] = v * wj
        pltpu.sync_copy(gbuf, acc.at[eidx.at[1, i]], add=True)
        return carry

    ncheck = jnp.where(c == 0, _NCH0, _NCH1)
    lax.fori_loop(0, ncheck, chunk, 0)
    plsc.subcore_barrier()
    _band_copy(s, acc, out.at[c])


@functools.cache
def _get_sc_linear():
    return pl.kernel(
        _sc_linear_body,
        out_type=jax.ShapeDtypeStruct((_NC, _D, _B), jnp.float32),
        mesh=plsc.VectorSubcoreMesh(core_axis_name="c", subcore_axis_name="s",
                                    num_cores=_NC, num_subcores=_NS),
        compiler_params=pltpu.CompilerParams(needs_layout_passes=False,
                                             use_tc_tiling_on_sc=False),
        scratch_types=[
            pltpu.VMEM_SHARED((_D, _B), jnp.float32),
            pltpu.VMEM((2, _MAXCH, _CH), jnp.int32),
            pltpu.VMEM((_MAXCH * _CH,), jnp.float32),
            pltpu.VMEM((_CH, _B), jnp.float32),
            pltpu.SemaphoreType.DMA,
        ],
    )


_RG = 10


def _tc_norm_body(p_ref, b_ref, g_ref, be_ref, o_ref):
    acc = p_ref[0] + p_ref[1]
    acc = acc + b_ref[0][:, :, None]
    mu = jnp.mean(acc, axis=1, keepdims=True)
    xc = acc - mu
    var = jnp.mean(xc * xc, axis=1, keepdims=True)
    y = xc * lax.rsqrt(var + _EPS)
    y = y * g_ref[0][:, :, None] + be_ref[0][:, :, None]
    o_ref[...] = jnp.maximum(y, 0.0)


_tc_norm = pl.pallas_call(
    _tc_norm_body,
    grid=(_G // _RG,),
    in_specs=[
        pl.BlockSpec((2, _RG, _GS, _B), lambda i: (0, i, 0, 0)),
        pl.BlockSpec((1, _RG, _GS), lambda i: (i, 0, 0)),
        pl.BlockSpec((1, _RG, _GS), lambda i: (i, 0, 0)),
        pl.BlockSpec((1, _RG, _GS), lambda i: (i, 0, 0)),
    ],
    out_specs=pl.BlockSpec((_RG, _GS, _B), lambda i: (i, 0, 0)),
    out_shape=jax.ShapeDtypeStruct((_G, _GS, _B), jnp.float32),
)

_RROW = 1000


def _tc_final_body(p_ref, b_ref, x_ref, o_ref):
    o_ref[...] = p_ref[0] + p_ref[1] + b_ref[...] + x_ref[...]


_tc_final = pl.pallas_call(
    _tc_final_body,
    grid=(_D // _RROW,),
    in_specs=[
        pl.BlockSpec((2, _RROW, _B), lambda i: (0, i, 0)),
        pl.BlockSpec((_RROW, 1), lambda i: (i, 0)),
        pl.BlockSpec((_RROW, _B), lambda i: (i, 0)),
    ],
    out_specs=pl.BlockSpec((_RROW, _B), lambda i: (i, 0)),
    out_shape=jax.ShapeDtypeStruct((_D, _B), jnp.float32),
)


def _split_uneven(flat):
    n0 = _NS * _NCH0 * _CH
    c0 = flat[:n0].reshape(_NS, _NCH0 * _CH)
    c0 = jnp.pad(c0, ((0, 0), (0, (_MAXCH - _NCH0) * _CH)))
    c1 = flat[n0:].reshape(_NS, _NCH1 * _CH)
    c1 = jnp.pad(c1, ((0, 0), (0, (_MAXCH - _NCH1) * _CH)))
    return jnp.concatenate([c0, c1], axis=0)


def _prep_edges(ei, w):
    pad = _EP - _E
    r = _split_uneven(jnp.pad(ei[0], (0, pad))).reshape(_NW, _MAXCH, _CH)
    c = _split_uneven(jnp.pad(ei[1], (0, pad))).reshape(_NW, _MAXCH, _CH)
    wp = _split_uneven(jnp.pad(w, (0, pad)))
    return r, c, wp


def kernel(x, batched_edge_indices1, batched_edge_indices2, batched_edge_indices3,
           w1, b1, gamma1, beta1, w2, b2, gamma2, beta2, w3, b3):
    xT = x.T
    zeros = jnp.zeros((_D, _B), jnp.float32)
    r1, c1, wp1 = _prep_edges(batched_edge_indices1, w1)
    r2, c2, wp2 = _prep_edges(batched_edge_indices2, w2)
    r3, c3, wp3 = _prep_edges(batched_edge_indices3, w3)

    _sc_linear = _get_sc_linear()
    _shp = (_G // _RG, _RG, _GS)
    p1 = _sc_linear(xT, c1, r1, wp1, zeros)
    h1 = _tc_norm(p1.reshape(_NC, _G, _GS, _B), b1.reshape(_shp),
                  gamma1.reshape(_shp), beta1.reshape(_shp)).reshape(_D, _B)
    p2 = _sc_linear(h1, c2, r2, wp2, zeros)
    h2 = _tc_norm(p2.reshape(_NC, _G, _GS, _B), b2.reshape(_shp),
                  gamma2.reshape(_shp), beta2.reshape(_shp)).reshape(_D, _B)
    p3 = _sc_linear(h2, c3, r3, wp3, zeros)
    outT = _tc_final(p3, b3.reshape(_D, 1), xT)
    return outT.T


# speedup vs baseline: 1.4223x; 1.2952x over previous
import functools

import jax
import jax.numpy as jnp
from jax import lax
from jax.experimental import pallas as pl
from jax.experimental.pallas import tpu as pltpu
from jax.experimental.pallas import tpu_sc as plsc

_B = 64
_D = 10000
_E = 320000
_G = 100
_GS = 100
_EPS = 1e-5

_NC = 2
_NS = 16
_NW = _NC * _NS
_CH = 128
_NCHUNK = 79
_EPT = _CH * _NCHUNK
_EP = _EPT * _NW
_BAND = 640
_TAIL = _D - 15 * _BAND


def _band_copy(s, src, dst):
    @pl.when(s < _NS - 1)
    def _():
        pltpu.sync_copy(src.at[pl.ds(s * _BAND, _BAND)],
                        dst.at[pl.ds(s * _BAND, _BAND)])

    @pl.when(s == _NS - 1)
    def _():
        pltpu.sync_copy(src.at[pl.ds(15 * _BAND, _TAIL)],
                        dst.at[pl.ds(15 * _BAND, _TAIL)])


def _sc_linear_body(table, cols, rows, wvals, zeros, out, acc, eidx, ew, gbuf, gsem):
    c = lax.axis_index("c")
    s = lax.axis_index("s")
    wid = c * _NS + s
    _band_copy(s, zeros, acc)
    pltpu.sync_copy(cols.at[wid], eidx.at[0])
    pltpu.sync_copy(rows.at[wid], eidx.at[1])
    pltpu.sync_copy(wvals.at[wid], ew)
    plsc.subcore_barrier()

    def chunk(i, carry):
        pltpu.async_copy(table.at[eidx.at[0, i]], gbuf, gsem).wait()
        base_vec = jnp.full((16,), 0, jnp.int32) + i * _CH
        for j in range(_CH):
            wj = plsc.load_gather(ew, [base_vec + j])
            for k in range(_B // 16):
                v = gbuf[j, pl.ds(k * 16, 16)]
                gbuf[j, pl.ds(k * 16, 16)] = v * wj
        pltpu.sync_copy(gbuf, acc.at[eidx.at[1, i]], add=True)
        return carry

    lax.fori_loop(0, _NCHUNK, chunk, 0)
    plsc.subcore_barrier()
    _band_copy(s, acc, out.at[c])


@functools.cache
def _get_sc_linear():
    return pl.kernel(
        _sc_linear_body,
        out_type=jax.ShapeDtypeStruct((_NC, _D, _B), jnp.float32),
        mesh=plsc.VectorSubcoreMesh(core_axis_name="c", subcore_axis_name="s",
                                    num_cores=_NC, num_subcores=_NS),
        compiler_params=pltpu.CompilerParams(needs_layout_passes=False,
                                             use_tc_tiling_on_sc=False),
        scratch_types=[
            pltpu.VMEM_SHARED((_D, _B), jnp.float32),
            pltpu.VMEM((2, _NCHUNK, _CH), jnp.int32),
            pltpu.VMEM((_EPT,), jnp.float32),
            pltpu.VMEM((_CH, _B), jnp.float32),
            pltpu.SemaphoreType.DMA,
        ],
    )


_RG = 10


def _tc_norm_body(p_ref, b_ref, g_ref, be_ref, o_ref):
    acc = p_ref[0] + p_ref[1]
    acc = acc + b_ref[0][:, :, None]
    mu = jnp.mean(acc, axis=1, keepdims=True)
    xc = acc - mu
    var = jnp.mean(xc * xc, axis=1, keepdims=True)
    y = xc * lax.rsqrt(var + _EPS)
    y = y * g_ref[0][:, :, None] + be_ref[0][:, :, None]
    o_ref[...] = jnp.maximum(y, 0.0)


_tc_norm = pl.pallas_call(
    _tc_norm_body,
    grid=(_G // _RG,),
    in_specs=[
        pl.BlockSpec((2, _RG, _GS, _B), lambda i: (0, i, 0, 0)),
        pl.BlockSpec((1, _RG, _GS), lambda i: (i, 0, 0)),
        pl.BlockSpec((1, _RG, _GS), lambda i: (i, 0, 0)),
        pl.BlockSpec((1, _RG, _GS), lambda i: (i, 0, 0)),
    ],
    out_specs=pl.BlockSpec((_RG, _GS, _B), lambda i: (i, 0, 0)),
    out_shape=jax.ShapeDtypeStruct((_G, _GS, _B), jnp.float32),
)

_RROW = 1000


def _tc_final_body(p_ref, b_ref, x_ref, o_ref):
    o_ref[...] = p_ref[0] + p_ref[1] + b_ref[...] + x_ref[...]


_tc_final = pl.pallas_call(
    _tc_final_body,
    grid=(_D // _RROW,),
    in_specs=[
        pl.BlockSpec((2, _RROW, _B), lambda i: (0, i, 0)),
        pl.BlockSpec((_RROW, 1), lambda i: (i, 0)),
        pl.BlockSpec((_RROW, _B), lambda i: (i, 0)),
    ],
    out_specs=pl.BlockSpec((_RROW, _B), lambda i: (i, 0)),
    out_shape=jax.ShapeDtypeStruct((_D, _B), jnp.float32),
)


def _prep_edges(ei, w):
    pad = _EP - _E
    r = jnp.pad(ei[0], (0, pad)).reshape(_NW, _NCHUNK, _CH)
    c = jnp.pad(ei[1], (0, pad)).reshape(_NW, _NCHUNK, _CH)
    wp = jnp.pad(w, (0, pad)).reshape(_NW, _EPT)
    return r, c, wp


def kernel(x, batched_edge_indices1, batched_edge_indices2, batched_edge_indices3,
           w1, b1, gamma1, beta1, w2, b2, gamma2, beta2, w3, b3):
    xT = x.T
    zeros = jnp.zeros((_D, _B), jnp.float32)
    r1, c1, wp1 = _prep_edges(batched_edge_indices1, w1)
    r2, c2, wp2 = _prep_edges(batched_edge_indices2, w2)
    r3, c3, wp3 = _prep_edges(batched_edge_indices3, w3)

    _sc_linear = _get_sc_linear()
    _shp = (_G // _RG, _RG, _GS)
    p1 = _sc_linear(xT, c1, r1, wp1, zeros)
    h1 = _tc_norm(p1.reshape(_NC, _G, _GS, _B), b1.reshape(_shp),
                  gamma1.reshape(_shp), beta1.reshape(_shp)).reshape(_D, _B)
    p2 = _sc_linear(h1, c2, r2, wp2, zeros)
    h2 = _tc_norm(p2.reshape(_NC, _G, _GS, _B), b2.reshape(_shp),
                  gamma2.reshape(_shp), beta2.reshape(_shp)).reshape(_D, _B)
    p3 = _sc_linear(h2, c3, r3, wp3, zeros)
    outT = _tc_final(p3, b3.reshape(_D, 1), xT)
    return outT.T
